# baseline (speedup 1.0000x reference)
import numpy as np
import jax
import jax.numpy as jnp
from jax.experimental import pallas as pl
from jax.experimental.pallas import tpu as pltpu

_NUM_LIMBS = 22
_S1BITS = sum(
    1 << v for v in (0, 1, 2, 3, 4, 5, 6, 8, 9, 10, 12, 13, 14, 16, 17, 18, 20, 21)
)
_MABITS = _S1BITS | (1 << 7) | (1 << 11) | (1 << 15) | (1 << 19)

_ROWS = 8


def _loss_kernel(gt_ref, pr_ref, out_ref, m1_s, m7_s, m11_s, m15_s, m19_s, ma_s, fin_s):
    i = pl.program_id(0)
    shape = gt_ref.shape

    @pl.when(i == 0)
    def _():
        lane = jax.lax.broadcasted_iota(jnp.int32, shape, 1)
        k = lane % 23
        pos69 = lane % 69

        def bit(bits, idx):
            return ((jnp.int32(bits) >> idx) & 1).astype(jnp.float32)

        m1_s[...] = bit(_S1BITS, k)
        m7_s[...] = (k == 7).astype(jnp.float32)
        m11_s[...] = (k == 11).astype(jnp.float32)
        m15_s[...] = (k == 15).astype(jnp.float32)
        m19_s[...] = (k == 19).astype(jnp.float32)
        ma_s[...] = bit(_MABITS, k)
        fin_s[...] = jnp.where(
            pos69 < 23, bit(_MABITS, jnp.minimum(pos69, 22)), jnp.float32(0)
        )
        out_ref[...] = jnp.zeros((1, 1), jnp.float32)

    g = gt_ref[...] - pr_ref[...]

    def sh(x, s):
        return pltpu.roll(x, (-s) % shape[1], axis=1)

    left = (
        m1_s[...] * g
        + m7_s[...] * sh(g, -4)
        + m11_s[...] * sh(g, -8)
        + m15_s[...] * sh(g, -15)
        + m19_s[...] * sh(g, -19)
    )
    f = left - ma_s[...] * sh(g, 1)
    f = f * f
    v = f + sh(f, 23) + sh(f, 46)
    part = jnp.sum(jnp.sqrt(v) * fin_s[...]).reshape(1, 1)
    out_ref[...] += part


def kernel(kpts_gt, kpts_pred):
    n, ncoord, nkpt = kpts_gt.shape
    nfeat = ncoord * nkpt
    width = nfeat * 128
    nrows = n * nfeat // width
    grid = nrows // _ROWS
    gt2 = kpts_gt.reshape(nrows, width)
    pr2 = kpts_pred.reshape(nrows, width)
    total = pl.pallas_call(
        _loss_kernel,
        grid=(grid,),
        in_specs=[
            pl.BlockSpec((_ROWS, width), lambda i: (i, 0)),
            pl.BlockSpec((_ROWS, width), lambda i: (i, 0)),
        ],
        out_specs=pl.BlockSpec((1, 1), lambda i: (0, 0)),
        out_shape=jax.ShapeDtypeStruct((1, 1), jnp.float32),
        scratch_shapes=[pltpu.VMEM((_ROWS, width), jnp.float32)] * 7,
    )(gt2, pr2)
    return total[0, 0] / np.float32(n * _NUM_LIMBS)


# speedup vs baseline: 4.1442x; 4.1442x over previous
import numpy as np
import jax
import jax.numpy as jnp
from jax.experimental import pallas as pl

_FROM = (0, 1, 2, 3, 4, 5, 6, 3, 8, 9, 10, 3, 12, 13, 14, 0, 16, 17, 18, 0, 20, 21)
_TO = tuple(range(1, 23))
_NUM_LIMBS = 22


def _selection_matrix() -> np.ndarray:
    sel = np.zeros((69, 128), dtype=np.float32)
    for c in range(3):
        for l in range(_NUM_LIMBS):
            sel[c * 23 + _FROM[l], 32 * c + l] += 1.0
            sel[c * 23 + _TO[l], 32 * c + l] -= 1.0
    return sel


def _loss_kernel(gt_ref, pr_ref, sel_ref, out_ref):
    i = pl.program_id(0)
    d = gt_ref[...] - pr_ref[...]
    y = jnp.dot(d, sel_ref[...], preferred_element_type=jnp.float32)
    sq = y * y
    v = sq[:, 0:32] + sq[:, 32:64] + sq[:, 64:96]
    part = jnp.sum(jnp.sqrt(v)).reshape(1, 1)

    @pl.when(i == 0)
    def _():
        out_ref[...] = jnp.zeros((1, 1), jnp.float32)

    out_ref[...] += part


def kernel(kpts_gt, kpts_pred):
    n, ncoord, nkpt = kpts_gt.shape
    nfeat = ncoord * nkpt
    block_b = 4096
    grid = n // block_b
    sel = jnp.asarray(_selection_matrix())
    gt2 = kpts_gt.reshape(n, nfeat)
    pr2 = kpts_pred.reshape(n, nfeat)
    total = pl.pallas_call(
        _loss_kernel,
        grid=(grid,),
        in_specs=[
            pl.BlockSpec((block_b, nfeat), lambda i: (i, 0)),
            pl.BlockSpec((block_b, nfeat), lambda i: (i, 0)),
            pl.BlockSpec((nfeat, 128), lambda i: (0, 0)),
        ],
        out_specs=pl.BlockSpec((1, 1), lambda i: (0, 0)),
        out_shape=jax.ShapeDtypeStruct((1, 1), jnp.float32),
    )(gt2, pr2, sel)
    return total[0, 0] / np.float32(n * _NUM_LIMBS)
